# Initial kernel scaffold; baseline (speedup 1.0000x reference)
import jax
import jax.numpy as jnp
from jax.experimental import pallas as pl


def kernel(x, duty_cycles):
    raise NotImplementedError("write your pallas kernel here")


# speedup vs baseline: 8.8788x; 8.8788x over previous
import jax
import jax.numpy as jnp
from jax import lax
from jax.experimental import pallas as pl
from jax.experimental.pallas import tpu as pltpu
from jax.experimental.pallas import tpu_sc as plsc

BATCH = 128
N = 32768
KSEL = 3277
NC = 2
NS = 16
NW = NC * NS
ROWS_PER_W = BATCH // NW
L = 16
NV = N // L
NBINS = 256
HIST_WORDS = L * NBINS


def _body(x_hbm, duty_hbm, out_hbm, x_v, keys_v, bf_v, hist_v):
    wid = lax.axis_index("s") * NC + lax.axis_index("c")
    lanes = lax.iota(jnp.int32, L)
    lane_off = lanes * NBINS
    ones = jnp.ones((L,), jnp.int32)
    zeros_i = jnp.zeros((L,), jnp.int32)

    pltpu.sync_copy(duty_hbm, x_v)
    td = jnp.float32(KSEL / N)

    def bf_step(i, c):
        d = x_v[pl.ds(i * L, L)]
        bf_v[pl.ds(i * L, L)] = jnp.exp(td - d)
        return c

    lax.fori_loop(0, NV, bf_step, 0)

    def zero_hist():
        def z_step(j, c):
            hist_v[pl.ds(j * L, L)] = zeros_i
            return c
        lax.fori_loop(0, HIST_WORDS // L, z_step, 0)

    def bin_search(r):
        def g_step(gi, carry):
            C, bsum, ssum = carry
            base = (15 - gi) * L
            v = hist_v[pl.ds(base, L)]
            for l in range(1, L):
                v = v + hist_v[pl.ds(l * NBINS + base, L)]
            rev = lax.rev(v, (0,))
            cs = plsc.cumsum(rev)
            up = C + cs
            s_strict = up - rev
            m = jnp.logical_and(s_strict < r, up >= r)
            binvec = (base + (L - 1)) - lanes
            bsum = bsum + jnp.sum(jnp.where(m, binvec, 0))
            ssum = ssum + jnp.sum(jnp.where(m, s_strict, 0))
            C = C + jnp.sum(v)
            return (C, bsum, ssum)

        C, bsum, ssum = lax.fori_loop(
            0, 16, g_step, (jnp.int32(0), jnp.int32(0), jnp.int32(0)))
        return bsum, r - ssum

    def row_step(ri, c):
        row = wid * ROWS_PER_W + ri
        pltpu.sync_copy(x_hbm.at[row], x_v)

        zero_hist()

        def p1_step(i, cc):
            sl = pl.ds(i * L, L)
            b = x_v[sl] * bf_v[sl]
            bi = lax.bitcast_convert_type(b, jnp.int32)
            t = lax.shift_right_arithmetic(bi, 31)
            key = lax.bitwise_xor(bi, lax.bitwise_and(t, jnp.int32(0x7FFFFFFF)))
            keys_v[sl] = key
            bucket = lax.shift_right_arithmetic(key, 24) + 128
            plsc.addupdate_scatter(hist_v, [lane_off + bucket], ones)
            return cc

        lax.fori_loop(0, NV, p1_step, 0)
        b1, r = bin_search(jnp.int32(KSEL))
        prefix = b1 - 128

        def radix_pass(shift_hi, shift_lo, prefix, r):
            zero_hist()

            def p_step(i, cc):
                key = keys_v[pl.ds(i * L, L)]
                active = lax.shift_right_arithmetic(key, shift_hi) == prefix
                bucket = lax.bitwise_and(
                    lax.shift_right_arithmetic(key, shift_lo), jnp.int32(255))
                plsc.addupdate_scatter(
                    hist_v, [lane_off + bucket], ones, mask=active)
                return cc

            lax.fori_loop(0, NV, p_step, 0)
            return bin_search(r)

        b2, r = radix_pass(24, 16, prefix, r)
        prefix = prefix * 256 + b2
        b3, r = radix_pass(16, 8, prefix, r)
        prefix = prefix * 256 + b3
        b4, r = radix_pass(8, 0, prefix, r)
        kth = prefix * 256 + b4

        def out_step(i, cc):
            sl = pl.ds(i * L, L)
            m = keys_v[sl] >= kth
            x_v[sl] = jnp.where(m, x_v[sl], jnp.float32(0.0))
            return cc

        lax.fori_loop(0, NV, out_step, 0)
        pltpu.sync_copy(x_v, out_hbm.at[row])
        return c

    lax.fori_loop(0, ROWS_PER_W, row_step, 0)


def kernel(x, duty_cycles):
    mesh = plsc.VectorSubcoreMesh(
        core_axis_name="c", subcore_axis_name="s",
        num_cores=NC, num_subcores=NS)
    f = pl.kernel(
        _body,
        out_type=jax.ShapeDtypeStruct((BATCH, N), jnp.float32),
        mesh=mesh,
        compiler_params=pltpu.CompilerParams(needs_layout_passes=False),
        scratch_types=[
            pltpu.VMEM((N,), jnp.float32),
            pltpu.VMEM((N,), jnp.int32),
            pltpu.VMEM((N,), jnp.float32),
            pltpu.VMEM((HIST_WORDS,), jnp.int32),
        ],
    )
    return f(x, duty_cycles)


# speedup vs baseline: 32.3341x; 3.6417x over previous
import jax
import jax.numpy as jnp
from jax import lax
from jax.experimental import pallas as pl
from jax.experimental.pallas import tpu as pltpu
from jax.experimental.pallas import tpu_sc as plsc

BATCH = 128
N = 32768
KSEL = 3277
NC = 2
NS = 16
NW = NC * NS
ROWS_PER_W = BATCH // NW
L = 16
NV = N // L
NBINS = 256
HIST_WORDS = L * NBINS


def _body(x_hbm, duty_hbm, out_hbm, x_v, keys_v, bf_v, hist_v):
    wid = lax.axis_index("s") * NC + lax.axis_index("c")
    lanes = lax.iota(jnp.int32, L)
    lane_off = lanes * NBINS
    ones = jnp.ones((L,), jnp.int32)
    zeros_i = jnp.zeros((L,), jnp.int32)

    pltpu.sync_copy(duty_hbm, x_v)
    td = jnp.float32(KSEL / N)

    @plsc.parallel_loop(0, N, L, unroll=8)
    def bf_step(i):
        bf_v[pl.ds(i, L)] = jnp.exp(td - x_v[pl.ds(i, L)])

    def zero_hist():
        @plsc.parallel_loop(0, HIST_WORDS, L, unroll=8)
        def z_step(j):
            hist_v[pl.ds(j, L)] = zeros_i

    def bin_search(r):
        def g_step(gi, carry):
            C, bsum, ssum = carry
            base = (15 - gi) * L
            v = hist_v[pl.ds(base, L)]
            for l in range(1, L):
                v = v + hist_v[pl.ds(l * NBINS + base, L)]
            rev = lax.rev(v, (0,))
            cs = plsc.cumsum(rev)
            up = C + cs
            s_strict = up - rev
            m = jnp.logical_and(s_strict < r, up >= r)
            binvec = (base + (L - 1)) - lanes
            bsum = bsum + jnp.sum(jnp.where(m, binvec, 0))
            ssum = ssum + jnp.sum(jnp.where(m, s_strict, 0))
            C = C + jnp.sum(v)
            return (C, bsum, ssum)

        C, bsum, ssum = lax.fori_loop(
            0, 16, g_step, (jnp.int32(0), jnp.int32(0), jnp.int32(0)))
        return bsum, r - ssum

    def row_step(ri, c):
        row = wid * ROWS_PER_W + ri
        pltpu.sync_copy(x_hbm.at[row], x_v)

        zero_hist()

        @plsc.parallel_loop(0, N, L, unroll=4)
        def p1_step(i):
            sl = pl.ds(i, L)
            b = x_v[sl] * bf_v[sl]
            bi = lax.bitcast_convert_type(b, jnp.int32)
            t = lax.shift_right_arithmetic(bi, 31)
            key = lax.bitwise_xor(bi, lax.bitwise_and(t, jnp.int32(0x7FFFFFFF)))
            keys_v[sl] = key
            bucket = lax.shift_right_arithmetic(key, 24) + 128
            plsc.addupdate_scatter(hist_v, [lane_off + bucket], ones)

        b1, r = bin_search(jnp.int32(KSEL))
        prefix = b1 - 128

        def radix_pass(shift_hi, shift_lo, prefix, r):
            zero_hist()

            @plsc.parallel_loop(0, N, L, unroll=4)
            def p_step(i):
                key = keys_v[pl.ds(i, L)]
                active = lax.shift_right_arithmetic(key, shift_hi) == prefix
                bucket = lax.bitwise_and(
                    lax.shift_right_arithmetic(key, shift_lo), jnp.int32(255))
                plsc.addupdate_scatter(
                    hist_v, [lane_off + bucket], ones, mask=active)

            return bin_search(r)

        b2, r = radix_pass(24, 16, prefix, r)
        prefix = prefix * 256 + b2
        b3, r = radix_pass(16, 8, prefix, r)
        prefix = prefix * 256 + b3
        b4, r = radix_pass(8, 0, prefix, r)
        kth = prefix * 256 + b4

        @plsc.parallel_loop(0, N, L, unroll=8)
        def out_step(i):
            sl = pl.ds(i, L)
            m = keys_v[sl] >= kth
            x_v[sl] = jnp.where(m, x_v[sl], jnp.float32(0.0))
        pltpu.sync_copy(x_v, out_hbm.at[row])
        return c

    lax.fori_loop(0, ROWS_PER_W, row_step, 0)


def kernel(x, duty_cycles):
    mesh = plsc.VectorSubcoreMesh(
        core_axis_name="c", subcore_axis_name="s",
        num_cores=NC, num_subcores=NS)
    f = pl.kernel(
        _body,
        out_type=jax.ShapeDtypeStruct((BATCH, N), jnp.float32),
        mesh=mesh,
        compiler_params=pltpu.CompilerParams(needs_layout_passes=False),
        scratch_types=[
            pltpu.VMEM((N,), jnp.float32),
            pltpu.VMEM((N,), jnp.int32),
            pltpu.VMEM((N,), jnp.float32),
            pltpu.VMEM((HIST_WORDS,), jnp.int32),
        ],
    )
    return f(x, duty_cycles)


# speedup vs baseline: 32.8244x; 1.0152x over previous
import jax
import jax.numpy as jnp
from jax import lax
from jax.experimental import pallas as pl
from jax.experimental.pallas import tpu as pltpu
from jax.experimental.pallas import tpu_sc as plsc

BATCH = 128
N = 32768
KSEL = 3277
NC = 2
NS = 16
NW = NC * NS
ROWS_PER_W = BATCH // NW
L = 16
NV = N // L
NBINS = 256
HIST_WORDS = L * NBINS


def _body(x_hbm, duty_hbm, out_hbm, x_v, keys_v, bf_v, hist_v):
    wid = lax.axis_index("s") * NC + lax.axis_index("c")
    lanes = lax.iota(jnp.int32, L)
    lane_off = lanes * NBINS
    ones = jnp.ones((L,), jnp.int32)
    zeros_i = jnp.zeros((L,), jnp.int32)

    pltpu.sync_copy(duty_hbm, x_v)
    td = jnp.float32(KSEL / N)

    @plsc.parallel_loop(0, N, L, unroll=8)
    def bf_step(i):
        bf_v[pl.ds(i, L)] = jnp.exp(td - x_v[pl.ds(i, L)])

    def zero_hist():
        @plsc.parallel_loop(0, HIST_WORDS, L, unroll=8)
        def z_step(j):
            hist_v[pl.ds(j, L)] = zeros_i

    def bin_search(r):
        def g_step(gi, carry):
            C, bsum, ssum = carry
            base = (15 - gi) * L
            v = hist_v[pl.ds(base, L)]
            for l in range(1, L):
                v = v + hist_v[pl.ds(l * NBINS + base, L)]
            rev = lax.rev(v, (0,))
            cs = plsc.cumsum(rev)
            up = C + cs
            s_strict = up - rev
            m = jnp.logical_and(s_strict < r, up >= r)
            binvec = (base + (L - 1)) - lanes
            bsum = bsum + jnp.sum(jnp.where(m, binvec, 0))
            ssum = ssum + jnp.sum(jnp.where(m, s_strict, 0))
            C = C + jnp.sum(v)
            return (C, bsum, ssum)

        C, bsum, ssum = lax.fori_loop(
            0, 16, g_step, (jnp.int32(0), jnp.int32(0), jnp.int32(0)))
        return bsum, r - ssum

    def row_step(ri, c):
        row = wid * ROWS_PER_W + ri
        pltpu.sync_copy(x_hbm.at[row], x_v)

        zero_hist()

        @plsc.parallel_loop(0, N, L, unroll=8)
        def p1_step(i):
            sl = pl.ds(i, L)
            b = x_v[sl] * bf_v[sl]
            bi = lax.bitcast_convert_type(b, jnp.int32)
            t = lax.shift_right_arithmetic(bi, 31)
            key = lax.bitwise_xor(bi, lax.bitwise_and(t, jnp.int32(0x7FFFFFFF)))
            keys_v[sl] = key
            bucket = lax.shift_right_arithmetic(key, 24) + 128
            plsc.addupdate_scatter(hist_v, [lane_off + bucket], ones)

        b1, r = bin_search(jnp.int32(KSEL))
        prefix = b1 - 128

        def radix_pass(shift_hi, shift_lo, prefix, r):
            zero_hist()

            @plsc.parallel_loop(0, N, L, unroll=8)
            def p_step(i):
                key = keys_v[pl.ds(i, L)]
                active = lax.shift_right_arithmetic(key, shift_hi) == prefix
                bucket = lax.bitwise_and(
                    lax.shift_right_arithmetic(key, shift_lo), jnp.int32(255))
                plsc.addupdate_scatter(
                    hist_v, [lane_off + bucket], ones, mask=active)

            return bin_search(r)

        b2, r = radix_pass(24, 16, prefix, r)
        prefix = prefix * 256 + b2
        b3, r = radix_pass(16, 8, prefix, r)
        prefix = prefix * 256 + b3
        b4, r = radix_pass(8, 0, prefix, r)
        kth = prefix * 256 + b4

        @plsc.parallel_loop(0, N, L, unroll=8)
        def out_step(i):
            sl = pl.ds(i, L)
            m = keys_v[sl] >= kth
            x_v[sl] = jnp.where(m, x_v[sl], jnp.float32(0.0))
        pltpu.sync_copy(x_v, out_hbm.at[row])
        return c

    lax.fori_loop(0, ROWS_PER_W, row_step, 0)


def kernel(x, duty_cycles):
    mesh = plsc.VectorSubcoreMesh(
        core_axis_name="c", subcore_axis_name="s",
        num_cores=NC, num_subcores=NS)
    f = pl.kernel(
        _body,
        out_type=jax.ShapeDtypeStruct((BATCH, N), jnp.float32),
        mesh=mesh,
        compiler_params=pltpu.CompilerParams(needs_layout_passes=False),
        scratch_types=[
            pltpu.VMEM((N,), jnp.float32),
            pltpu.VMEM((N,), jnp.int32),
            pltpu.VMEM((N,), jnp.float32),
            pltpu.VMEM((HIST_WORDS,), jnp.int32),
        ],
    )
    return f(x, duty_cycles)


# speedup vs baseline: 36.9081x; 1.1244x over previous
import jax
import jax.numpy as jnp
from jax import lax
from jax.experimental import pallas as pl
from jax.experimental.pallas import tpu as pltpu
from jax.experimental.pallas import tpu_sc as plsc

BATCH = 128
N = 32768
KSEL = 3277
NC = 2
NS = 16
NW = NC * NS
ROWS_PER_W = BATCH // NW
L = 16
NV = N // L
NBINS = 256
HIST_STRIDE = NBINS + 1
HIST_WORDS = L * HIST_STRIDE


def _body(x_hbm, duty_hbm, out_hbm, x_v, keys_v, bf_v, hist_v):
    wid = lax.axis_index("s") * NC + lax.axis_index("c")
    lanes = lax.iota(jnp.int32, L)
    lane_off = lanes * HIST_STRIDE
    ones = jnp.ones((L,), jnp.int32)
    zeros_i = jnp.zeros((L,), jnp.int32)

    pltpu.sync_copy(duty_hbm, x_v)
    td = jnp.float32(KSEL / N)

    @plsc.parallel_loop(0, N, L, unroll=8)
    def bf_step(i):
        bf_v[pl.ds(i, L)] = jnp.exp(td - x_v[pl.ds(i, L)])

    def zero_hist():
        @plsc.parallel_loop(0, HIST_WORDS, L, unroll=8)
        def z_step(j):
            hist_v[pl.ds(j, L)] = zeros_i

    def bin_search(r):
        def g_step(gi, carry):
            C, bsum, ssum = carry
            base = (15 - gi) * L
            v = hist_v[pl.ds(base, L)]
            for l in range(1, L):
                v = v + hist_v[pl.ds(l * HIST_STRIDE + base, L)]
            rev = lax.rev(v, (0,))
            cs = plsc.cumsum(rev)
            up = C + cs
            s_strict = up - rev
            m = jnp.logical_and(s_strict < r, up >= r)
            binvec = (base + (L - 1)) - lanes
            bsum = bsum + jnp.sum(jnp.where(m, binvec, 0))
            ssum = ssum + jnp.sum(jnp.where(m, s_strict, 0))
            C = C + jnp.sum(v)
            return (C, bsum, ssum)

        C, bsum, ssum = lax.fori_loop(
            0, 16, g_step, (jnp.int32(0), jnp.int32(0), jnp.int32(0)))
        return bsum, r - ssum

    def row_step(ri, c):
        row = wid * ROWS_PER_W + ri
        pltpu.sync_copy(x_hbm.at[row], x_v)

        zero_hist()

        @plsc.parallel_loop(0, N, L, unroll=8)
        def p1_step(i):
            sl = pl.ds(i, L)
            b = x_v[sl] * bf_v[sl]
            bi = lax.bitcast_convert_type(b, jnp.int32)
            t = lax.shift_right_arithmetic(bi, 31)
            key = lax.bitwise_xor(bi, lax.bitwise_and(t, jnp.int32(0x7FFFFFFF)))
            keys_v[sl] = key
            bucket = lax.shift_right_arithmetic(key, 24) + 128
            plsc.addupdate_scatter(hist_v, [lane_off + bucket], ones)

        b1, r = bin_search(jnp.int32(KSEL))
        prefix = b1 - 128

        def radix_pass(shift_hi, shift_lo, prefix, r):
            zero_hist()

            @plsc.parallel_loop(0, N, L, unroll=8)
            def p_step(i):
                key = keys_v[pl.ds(i, L)]
                active = lax.shift_right_arithmetic(key, shift_hi) == prefix
                bucket = lax.bitwise_and(
                    lax.shift_right_arithmetic(key, shift_lo), jnp.int32(255))
                plsc.addupdate_scatter(
                    hist_v, [lane_off + bucket], ones, mask=active)

            return bin_search(r)

        b2, r = radix_pass(24, 16, prefix, r)
        prefix = prefix * 256 + b2
        b3, r = radix_pass(16, 8, prefix, r)
        prefix = prefix * 256 + b3
        b4, r = radix_pass(8, 0, prefix, r)
        kth = prefix * 256 + b4

        @plsc.parallel_loop(0, N, L, unroll=8)
        def out_step(i):
            sl = pl.ds(i, L)
            m = keys_v[sl] >= kth
            x_v[sl] = jnp.where(m, x_v[sl], jnp.float32(0.0))
        pltpu.sync_copy(x_v, out_hbm.at[row])
        return c

    lax.fori_loop(0, ROWS_PER_W, row_step, 0)


def kernel(x, duty_cycles):
    mesh = plsc.VectorSubcoreMesh(
        core_axis_name="c", subcore_axis_name="s",
        num_cores=NC, num_subcores=NS)
    f = pl.kernel(
        _body,
        out_type=jax.ShapeDtypeStruct((BATCH, N), jnp.float32),
        mesh=mesh,
        compiler_params=pltpu.CompilerParams(needs_layout_passes=False),
        scratch_types=[
            pltpu.VMEM((N,), jnp.float32),
            pltpu.VMEM((N,), jnp.int32),
            pltpu.VMEM((N,), jnp.float32),
            pltpu.VMEM((HIST_WORDS,), jnp.int32),
        ],
    )
    return f(x, duty_cycles)
